# baseline (speedup 1.0000x reference)
import functools

import jax
import jax.numpy as jnp
from jax import lax
from jax.experimental import pallas as pl
from jax.experimental.pallas import tpu as pltpu
from jax.experimental.pallas import tpu_sc as plsc

_NC = 2
_NS = 16
_NW = _NC * _NS
_LANES = 16

_D = 128
_K = 6


def _make_gather_sum(n_out, block, d, k, chunk, per_w0, per_w1):
  assert _NS * (per_w0 + per_w1) == n_out
  nblk0, nblk1 = per_w0 // block, per_w1 // block
  assert nblk0 * block == per_w0 and nblk1 * block == per_w1
  assert nblk0 % 2 == 0 and nblk1 % 2 == 0
  ib = block * k
  nchunk = ib // chunk
  assert nchunk * chunk == ib and chunk <= 128 and chunk % 8 == 0
  mesh = plsc.VectorSubcoreMesh(core_axis_name="c", subcore_axis_name="s")

  @functools.partial(
      pl.kernel,
      mesh=mesh,
      out_type=jax.ShapeDtypeStruct((n_out, d), jnp.float32),
      scratch_types=[
          pltpu.VMEM((ib,), jnp.int32),
          pltpu.VMEM((ib,), jnp.int32),
          pltpu.VMEM((ib, d), jnp.float32),
          pltpu.VMEM((ib, d), jnp.float32),
          pltpu.VMEM((block, d), jnp.float32),
          pltpu.VMEM((block, d), jnp.float32),
          pltpu.SemaphoreType.DMA,
          pltpu.SemaphoreType.DMA,
      ],
  )
  def gsum(src_hbm, idx_hbm, out_hbm, idx0, idx1, rows0, rows1, acc0, acc1,
           sem_g, sem_o):
    c = lax.axis_index("c")
    s = lax.axis_index("s")
    out0 = jnp.where(c == 0, s * per_w0, _NS * per_w0 + s * per_w1)
    npair = jnp.where(c == 0, nblk0 // 2, nblk1 // 2)

    def fetch(idx_v, rows_v, b):
      pltpu.sync_copy(idx_hbm.at[pl.ds((out0 + b * block) * k, ib)], idx_v)
      for g in range(nchunk):
        pltpu.async_copy(
            src_hbm.at[idx_v.at[pl.ds(g * chunk, chunk)]],
            rows_v.at[pl.ds(g * chunk, chunk)],
            sem_g,
        )

    def reduce_block(rows_v, acc_v):
      @plsc.parallel_loop(0, block, unroll=4)
      def sum_row(i):
        base = i * k
        for c in range(d // _LANES):
          sl = pl.ds(c * _LANES, _LANES)
          acc = rows_v[base, sl]
          for j in range(1, k):
            acc = acc + rows_v[base + j, sl]
          acc_v[i, sl] = acc

    def drain_gather(rows_v):
      pltpu.make_async_copy(src_hbm.at[pl.ds(0, ib)], rows_v, sem_g).wait()

    def drain_out(acc_v):
      pltpu.make_async_copy(acc_v, out_hbm.at[pl.ds(out0, block)],
                            sem_o).wait()

    fetch(idx0, rows0, 0)

    def pair_body(p, _):
      b = 2 * p
      drain_gather(rows0)
      fetch(idx1, rows1, b + 1)
      @pl.when(p >= 1)
      def _():
        drain_out(acc0)
      reduce_block(rows0, acc0)
      pltpu.async_copy(acc0, out_hbm.at[pl.ds(out0 + b * block, block)],
                       sem_o)
      drain_gather(rows1)
      @pl.when(p + 1 < npair)
      def _():
        fetch(idx0, rows0, b + 2)
      @pl.when(p >= 1)
      def _():
        drain_out(acc1)
      reduce_block(rows1, acc1)
      pltpu.async_copy(acc1, out_hbm.at[pl.ds(out0 + (b + 1) * block, block)],
                       sem_o)
      return 0

    lax.fori_loop(0, npair, pair_body, 0, unroll=False)
    drain_out(acc0)
    drain_out(acc1)

  return gsum


def _mm_relu(x, w, bm):
  m, kdim = x.shape
  n = w.shape[1]

  def body(x_ref, w_ref, lin_ref, msg_ref):
    lin = jnp.dot(x_ref[...], w_ref[...], preferred_element_type=jnp.float32)
    lin_ref[...] = lin
    msg_ref[...] = jnp.maximum(lin, 0.0)

  return pl.pallas_call(
      body,
      grid=(m // bm,),
      in_specs=[
          pl.BlockSpec((bm, kdim), lambda i: (i, 0)),
          pl.BlockSpec((kdim, n), lambda i: (0, 0)),
      ],
      out_specs=[
          pl.BlockSpec((bm, n), lambda i: (i, 0)),
          pl.BlockSpec((bm, n), lambda i: (i, 0)),
      ],
      out_shape=[
          jax.ShapeDtypeStruct((m, n), jnp.float32),
          jax.ShapeDtypeStruct((m, n), jnp.float32),
      ],
  )(x, w)


def _update(nei, binput, w, bm):
  m, n = nei.shape

  def body(nei_ref, bin_ref, w_ref, out_ref):
    h = jnp.dot(nei_ref[...], w_ref[...], preferred_element_type=jnp.float32)
    out_ref[...] = jnp.maximum(bin_ref[...] + h, 0.0)

  return pl.pallas_call(
      body,
      grid=(m // bm,),
      in_specs=[
          pl.BlockSpec((bm, n), lambda i: (i, 0)),
          pl.BlockSpec((bm, n), lambda i: (i, 0)),
          pl.BlockSpec((n, n), lambda i: (0, 0)),
      ],
      out_specs=pl.BlockSpec((bm, n), lambda i: (i, 0)),
      out_shape=jax.ShapeDtypeStruct((m, n), jnp.float32),
  )(nei, binput, w)


def _atom_layer(fatoms, nei, w_o, b_o):
  m, da = fatoms.shape
  n = w_o.shape[1]

  def body(fa_ref, nei_ref, wo_ref, bo_ref, out_ref):
    h = jnp.dot(fa_ref[...], wo_ref[0:da, :], preferred_element_type=jnp.float32)
    h = h + jnp.dot(nei_ref[...], wo_ref[da:, :],
                    preferred_element_type=jnp.float32)
    out_ref[...] = jnp.maximum(h + bo_ref[...], 0.0)

  return pl.pallas_call(
      body,
      out_shape=jax.ShapeDtypeStruct((m, n), jnp.float32),
  )(fatoms, nei, w_o, b_o)


def kernel(fatoms, fbonds, agraph, bgraph, W_i, W_h, W_o, b_o):
  n_atoms, _ = fatoms.shape
  n_bonds, _ = fbonds.shape
  depth = 3

  bond_blk = 40
  atom_blk = 32
  a_pad = -(-n_atoms // (_NW * atom_blk)) * (_NW * atom_blk)

  fatoms_p = jnp.pad(fatoms, ((0, a_pad - n_atoms), (0, 0)))
  bidx = bgraph.reshape(-1)
  aidx = jnp.pad(agraph.reshape(-1), (0, (a_pad - n_atoms) * _K))

  gsum_bond = _make_gather_sum(n_bonds, bond_blk, _D, _K, chunk=80,
                               per_w0=12800, per_w1=7200)
  gsum_atom = _make_gather_sum(a_pad, atom_blk, _D, _K, chunk=96,
                               per_w0=384, per_w1=256)

  binput, message = _mm_relu(fbonds, W_i, bm=2000)
  for _ in range(depth - 1):
    nei = gsum_bond(message, bidx)
    message = _update(nei, binput, W_h, bm=2000)
  nei_atoms = gsum_atom(message, aidx)
  atom_h = _atom_layer(fatoms_p, nei_atoms, W_o, b_o.reshape(1, -1))
  return atom_h[:n_atoms]


# speedup vs baseline: 1.1342x; 1.1342x over previous
import functools

import jax
import jax.numpy as jnp
from jax import lax
from jax.experimental import pallas as pl
from jax.experimental.pallas import tpu as pltpu
from jax.experimental.pallas import tpu_sc as plsc

_NC = 2
_NS = 16
_NW = _NC * _NS
_LANES = 16

_D = 128
_K = 6


def _make_gather_sum(n_out, block, d, k, chunk, per_w0, per_w1):
  assert _NS * (per_w0 + per_w1) == n_out
  nblk0, nblk1 = per_w0 // block, per_w1 // block
  assert nblk0 * block == per_w0 and nblk1 * block == per_w1
  assert nblk0 % 2 == 0 and nblk1 % 2 == 0
  ib = block * k
  nchunk = ib // chunk
  assert nchunk * chunk == ib and chunk <= 128 and chunk % 8 == 0
  mesh = plsc.VectorSubcoreMesh(core_axis_name="c", subcore_axis_name="s")

  @functools.partial(
      pl.kernel,
      mesh=mesh,
      out_type=jax.ShapeDtypeStruct((n_out, d), jnp.float32),
      scratch_types=[
          pltpu.VMEM((ib,), jnp.int32),
          pltpu.VMEM((ib,), jnp.int32),
          pltpu.VMEM((ib, d), jnp.float32),
          pltpu.VMEM((ib, d), jnp.float32),
          pltpu.VMEM((block, d), jnp.float32),
          pltpu.VMEM((block, d), jnp.float32),
          pltpu.SemaphoreType.DMA,
          pltpu.SemaphoreType.DMA,
      ],
  )
  def gsum(src_hbm, idx_hbm, out_hbm, idx0, idx1, rows0, rows1, acc0, acc1,
           sem_g, sem_o):
    c = lax.axis_index("c")
    s = lax.axis_index("s")
    out0 = jnp.where(c == 0, s * per_w0, _NS * per_w0 + s * per_w1)
    npair = jnp.where(c == 0, nblk0 // 2, nblk1 // 2)

    def fetch(idx_v, rows_v, b):
      pltpu.sync_copy(idx_hbm.at[pl.ds((out0 + b * block) * k, ib)], idx_v)
      for g in range(nchunk):
        pltpu.async_copy(
            src_hbm.at[idx_v.at[pl.ds(g * chunk, chunk)]],
            rows_v.at[pl.ds(g * chunk, chunk)],
            sem_g,
        )

    def reduce_block(rows_v, acc_v):
      @plsc.parallel_loop(0, block, unroll=4)
      def sum_row(i):
        base = i * k
        for c in range(d // _LANES):
          sl = pl.ds(c * _LANES, _LANES)
          acc = rows_v[base, sl]
          for j in range(1, k):
            acc = acc + rows_v[base + j, sl]
          acc_v[i, sl] = acc

    def drain_gather(rows_v):
      pltpu.make_async_copy(src_hbm.at[pl.ds(0, ib)], rows_v, sem_g).wait()

    def drain_out(acc_v):
      pltpu.make_async_copy(acc_v, out_hbm.at[pl.ds(out0, block)],
                            sem_o).wait()

    fetch(idx0, rows0, 0)

    def pair_body(p, _):
      b = 2 * p
      drain_gather(rows0)
      fetch(idx1, rows1, b + 1)
      @pl.when(p >= 1)
      def _():
        drain_out(acc0)
      reduce_block(rows0, acc0)
      pltpu.async_copy(acc0, out_hbm.at[pl.ds(out0 + b * block, block)],
                       sem_o)
      drain_gather(rows1)
      @pl.when(p + 1 < npair)
      def _():
        fetch(idx0, rows0, b + 2)
      @pl.when(p >= 1)
      def _():
        drain_out(acc1)
      reduce_block(rows1, acc1)
      pltpu.async_copy(acc1, out_hbm.at[pl.ds(out0 + (b + 1) * block, block)],
                       sem_o)
      return 0

    lax.fori_loop(0, npair, pair_body, 0, unroll=False)
    drain_out(acc0)
    drain_out(acc1)

  return gsum


def _mm_relu(x, w, bm):
  m, kdim = x.shape
  n = w.shape[1]

  def body(x_ref, w_ref, lin_ref, msg_ref):
    lin = jnp.dot(x_ref[...], w_ref[...], preferred_element_type=jnp.float32)
    lin_ref[...] = lin
    msg_ref[...] = jnp.maximum(lin, 0.0)

  return pl.pallas_call(
      body,
      grid=(m // bm,),
      in_specs=[
          pl.BlockSpec((bm, kdim), lambda i: (i, 0)),
          pl.BlockSpec((kdim, n), lambda i: (0, 0)),
      ],
      out_specs=[
          pl.BlockSpec((bm, n), lambda i: (i, 0)),
          pl.BlockSpec((bm, n), lambda i: (i, 0)),
      ],
      out_shape=[
          jax.ShapeDtypeStruct((m, n), jnp.float32),
          jax.ShapeDtypeStruct((m, n), jnp.float32),
      ],
  )(x, w)


def _update(nei, binput, w, bm):
  m, n = nei.shape

  def body(nei_ref, bin_ref, w_ref, out_ref):
    h = jnp.dot(nei_ref[...], w_ref[...], preferred_element_type=jnp.float32)
    out_ref[...] = jnp.maximum(bin_ref[...] + h, 0.0)

  return pl.pallas_call(
      body,
      grid=(m // bm,),
      in_specs=[
          pl.BlockSpec((bm, n), lambda i: (i, 0)),
          pl.BlockSpec((bm, n), lambda i: (i, 0)),
          pl.BlockSpec((n, n), lambda i: (0, 0)),
      ],
      out_specs=pl.BlockSpec((bm, n), lambda i: (i, 0)),
      out_shape=jax.ShapeDtypeStruct((m, n), jnp.float32),
  )(nei, binput, w)


def _atom_layer(fatoms, nei, w_o, b_o):
  m, da = fatoms.shape
  n = w_o.shape[1]

  def body(fa_ref, nei_ref, wo_ref, bo_ref, out_ref):
    h = jnp.dot(fa_ref[...], wo_ref[0:da, :], preferred_element_type=jnp.float32)
    h = h + jnp.dot(nei_ref[...], wo_ref[da:, :],
                    preferred_element_type=jnp.float32)
    out_ref[...] = jnp.maximum(h + bo_ref[...], 0.0)

  return pl.pallas_call(
      body,
      out_shape=jax.ShapeDtypeStruct((m, n), jnp.float32),
  )(fatoms, nei, w_o, b_o)


def kernel(fatoms, fbonds, agraph, bgraph, W_i, W_h, W_o, b_o):
  n_atoms, _ = fatoms.shape
  n_bonds, _ = fbonds.shape
  depth = 3

  bond_blk = 40
  atom_blk = 32
  a_pad = -(-n_atoms // (_NW * atom_blk)) * (_NW * atom_blk)

  fatoms_p = jnp.pad(fatoms, ((0, a_pad - n_atoms), (0, 0)))
  bidx = bgraph.reshape(-1)
  aidx = jnp.pad(agraph.reshape(-1), (0, (a_pad - n_atoms) * _K))

  gsum_bond = _make_gather_sum(n_bonds, bond_blk, _D, _K, chunk=80,
                               per_w0=10000, per_w1=10000)
  gsum_atom = _make_gather_sum(a_pad, atom_blk, _D, _K, chunk=96,
                               per_w0=320, per_w1=320)

  binput, message = _mm_relu(fbonds, W_i, bm=2000)
  for _ in range(depth - 1):
    nei = gsum_bond(message, bidx)
    message = _update(nei, binput, W_h, bm=2000)
  nei_atoms = gsum_atom(message, aidx)
  atom_h = _atom_layer(fatoms_p, nei_atoms, W_o, b_o.reshape(1, -1))
  return atom_h[:n_atoms]


# speedup vs baseline: 1.4846x; 1.3089x over previous
import functools

import jax
import jax.numpy as jnp
from jax import lax
from jax.experimental import pallas as pl
from jax.experimental.pallas import tpu as pltpu
from jax.experimental.pallas import tpu_sc as plsc

_NC = 2
_NS = 16
_NW = _NC * _NS
_LANES = 16

_D = 128
_K = 6


def _make_gather_sum(n_out, block, d, k, chunk):
  per_w = n_out // _NW
  nblk = per_w // block
  assert per_w * _NW == n_out and nblk * block == per_w
  assert nblk % 2 == 0, "block count per worker must be even"
  npair = nblk // 2
  ib = block * k
  nchunk = ib // chunk
  assert nchunk * chunk == ib and chunk <= 128 and chunk % 8 == 0
  mesh = plsc.VectorSubcoreMesh(core_axis_name="c", subcore_axis_name="s")

  @functools.partial(
      pl.kernel,
      mesh=mesh,
      out_type=jax.ShapeDtypeStruct((n_out, d), jnp.float32),
      scratch_types=[
          pltpu.VMEM((ib,), jnp.int32),
          pltpu.VMEM((ib,), jnp.int32),
          pltpu.VMEM((ib, d), jnp.float32),
          pltpu.VMEM((ib, d), jnp.float32),
          pltpu.VMEM((block, d), jnp.float32),
          pltpu.VMEM((block, d), jnp.float32),
          pltpu.SemaphoreType.DMA,
          pltpu.SemaphoreType.DMA,
          pltpu.SemaphoreType.DMA,
      ],
  )
  def gsum(src_hbm, idx_hbm, out_hbm, idx0, idx1, rows0, rows1, acc0, acc1,
           sem_i, sem_g, sem_o):
    wid = lax.axis_index("s") * _NC + lax.axis_index("c")
    out0 = wid * per_w

    def fetch_idx(idx_v, b):
      base = out0 + b * block
      for j in range(k):
        pltpu.async_copy(idx_hbm.at[pl.ds(j * n_out + base, block)],
                         idx_v.at[pl.ds(j * block, block)], sem_i)

    def drain_idx(idx_v):
      pltpu.make_async_copy(idx_hbm.at[pl.ds(0, ib)], idx_v, sem_i).wait()

    def fetch_gather(idx_v, rows_v):
      for g in range(nchunk):
        pltpu.async_copy(
            src_hbm.at[idx_v.at[pl.ds(g * chunk, chunk)]],
            rows_v.at[pl.ds(g * chunk, chunk)],
            sem_g,
        )

    def drain_gather(rows_v):
      pltpu.make_async_copy(src_hbm.at[pl.ds(0, ib)], rows_v, sem_g).wait()

    def drain_out(acc_v):
      pltpu.make_async_copy(acc_v, out_hbm.at[pl.ds(out0, block)],
                            sem_o).wait()

    def reduce_block(rows_v, acc_v):
      @plsc.parallel_loop(0, block, unroll=4)
      def sum_row(i):
        for c in range(d // _LANES):
          sl = pl.ds(c * _LANES, _LANES)
          acc = rows_v[i, sl]
          for j in range(1, k):
            acc = acc + rows_v[j * block + i, sl]
          acc_v[i, sl] = acc

    fetch_idx(idx0, 0)
    drain_idx(idx0)
    fetch_gather(idx0, rows0)
    fetch_idx(idx1, 1)

    def pair_body(p, _):
      b = 2 * p
      drain_gather(rows0)
      drain_idx(idx1)
      fetch_gather(idx1, rows1)
      @pl.when(p + 1 < npair)
      def _():
        fetch_idx(idx0, b + 2)
      @pl.when(p >= 1)
      def _():
        drain_out(acc0)
      reduce_block(rows0, acc0)
      pltpu.async_copy(acc0, out_hbm.at[pl.ds(out0 + b * block, block)],
                       sem_o)
      drain_gather(rows1)
      @pl.when(p + 1 < npair)
      def _():
        drain_idx(idx0)
        fetch_gather(idx0, rows0)
        fetch_idx(idx1, b + 3)
      @pl.when(p >= 1)
      def _():
        drain_out(acc1)
      reduce_block(rows1, acc1)
      pltpu.async_copy(acc1, out_hbm.at[pl.ds(out0 + (b + 1) * block, block)],
                       sem_o)
      return 0

    lax.fori_loop(0, npair, pair_body, 0, unroll=False)
    drain_out(acc0)
    drain_out(acc1)

  return gsum


def _mm_relu_t(xt, w, bm):
  kdim, m = xt.shape
  n = w.shape[1]

  def body(x_ref, w_ref, lin_ref, msg_ref):
    lin = jax.lax.dot_general(x_ref[...], w_ref[...],
                              (((0,), (0,)), ((), ())),
                              preferred_element_type=jnp.float32)
    lin_ref[...] = lin
    msg_ref[...] = jnp.maximum(lin, 0.0)

  return pl.pallas_call(
      body,
      grid=(m // bm,),
      in_specs=[
          pl.BlockSpec((kdim, bm), lambda i: (0, i)),
          pl.BlockSpec((kdim, n), lambda i: (0, 0)),
      ],
      out_specs=[
          pl.BlockSpec((bm, n), lambda i: (i, 0)),
          pl.BlockSpec((bm, n), lambda i: (i, 0)),
      ],
      out_shape=[
          jax.ShapeDtypeStruct((m, n), jnp.float32),
          jax.ShapeDtypeStruct((m, n), jnp.float32),
      ],
  )(xt, w)


def _update(nei, binput, w, bm):
  m, n = nei.shape

  def body(nei_ref, bin_ref, w_ref, out_ref):
    h = jnp.dot(nei_ref[...], w_ref[...], preferred_element_type=jnp.float32)
    out_ref[...] = jnp.maximum(bin_ref[...] + h, 0.0)

  return pl.pallas_call(
      body,
      grid=(m // bm,),
      in_specs=[
          pl.BlockSpec((bm, n), lambda i: (i, 0)),
          pl.BlockSpec((bm, n), lambda i: (i, 0)),
          pl.BlockSpec((n, n), lambda i: (0, 0)),
      ],
      out_specs=pl.BlockSpec((bm, n), lambda i: (i, 0)),
      out_shape=jax.ShapeDtypeStruct((m, n), jnp.float32),
  )(nei, binput, w)


def _atom_layer(fatoms, nei, w_o, b_o):
  m, da = fatoms.shape
  n = w_o.shape[1]

  def body(fa_ref, nei_ref, wo_ref, bo_ref, out_ref):
    h = jnp.dot(fa_ref[...], wo_ref[0:da, :], preferred_element_type=jnp.float32)
    h = h + jnp.dot(nei_ref[...], wo_ref[da:, :],
                    preferred_element_type=jnp.float32)
    out_ref[...] = jnp.maximum(h + bo_ref[...], 0.0)

  return pl.pallas_call(
      body,
      out_shape=jax.ShapeDtypeStruct((m, n), jnp.float32),
  )(fatoms, nei, w_o, b_o)


def kernel(fatoms, fbonds, agraph, bgraph, W_i, W_h, W_o, b_o):
  n_atoms, _ = fatoms.shape
  n_bonds, _ = fbonds.shape
  depth = 3

  bond_blk = 40
  atom_blk = 32
  a_pad = -(-n_atoms // (_NW * atom_blk)) * (_NW * atom_blk)

  fatoms_p = jnp.pad(fatoms, ((0, a_pad - n_atoms), (0, 0)))
  bidx = bgraph.T.reshape(-1)
  aidx = jnp.pad(agraph, ((0, a_pad - n_atoms), (0, 0))).T.reshape(-1)

  gsum_bond = _make_gather_sum(n_bonds, bond_blk, _D, _K, chunk=80)
  gsum_atom = _make_gather_sum(a_pad, atom_blk, _D, _K, chunk=96)

  binput, message = _mm_relu_t(fbonds.T, W_i, bm=1280)
  for _ in range(depth - 1):
    nei = gsum_bond(message, bidx)
    message = _update(nei, binput, W_h, bm=2000)
  nei_atoms = gsum_atom(message, aidx)
  atom_h = _atom_layer(fatoms_p, nei_atoms, W_o, b_o.reshape(1, -1))
  return atom_h[:n_atoms]


# speedup vs baseline: 1.4964x; 1.0080x over previous
import functools

import jax
import jax.numpy as jnp
from jax import lax
from jax.experimental import pallas as pl
from jax.experimental.pallas import tpu as pltpu
from jax.experimental.pallas import tpu_sc as plsc

_NC = 2
_NS = 16
_NW = _NC * _NS
_LANES = 16

_D = 128
_K = 6


def _make_gather_sum(n_out, block, d, k, chunk):
  per_w = n_out // _NW
  nblk = per_w // block
  assert per_w * _NW == n_out and nblk * block == per_w
  assert nblk % 2 == 0, "block count per worker must be even"
  npair = nblk // 2
  ib = block * k
  nchunk = ib // chunk
  assert nchunk * chunk == ib and chunk <= 128 and chunk % 8 == 0
  mesh = plsc.VectorSubcoreMesh(core_axis_name="c", subcore_axis_name="s")

  @functools.partial(
      pl.kernel,
      mesh=mesh,
      out_type=jax.ShapeDtypeStruct((n_out, d), jnp.float32),
      scratch_types=[
          pltpu.VMEM((ib,), jnp.int32),
          pltpu.VMEM((ib,), jnp.int32),
          pltpu.VMEM((ib, d), jnp.float32),
          pltpu.VMEM((ib, d), jnp.float32),
          pltpu.VMEM((block, d), jnp.float32),
          pltpu.VMEM((block, d), jnp.float32),
          pltpu.SemaphoreType.DMA,
          pltpu.SemaphoreType.DMA,
          pltpu.SemaphoreType.DMA,
      ],
  )
  def gsum(src_hbm, idx_hbm, out_hbm, idx0, idx1, rows0, rows1, acc0, acc1,
           sem_i, sem_g, sem_o):
    wid = lax.axis_index("s") * _NC + lax.axis_index("c")
    out0 = wid * per_w

    def fetch_idx(idx_v, b):
      base = out0 + b * block
      for j in range(k):
        pltpu.async_copy(idx_hbm.at[pl.ds(j * n_out + base, block)],
                         idx_v.at[pl.ds(j * block, block)], sem_i)

    def drain_idx(idx_v):
      pltpu.make_async_copy(idx_hbm.at[pl.ds(0, ib)], idx_v, sem_i).wait()

    def fetch_gather(idx_v, rows_v):
      for g in range(nchunk):
        pltpu.async_copy(
            src_hbm.at[idx_v.at[pl.ds(g * chunk, chunk)]],
            rows_v.at[pl.ds(g * chunk, chunk)],
            sem_g,
        )

    def drain_gather(rows_v):
      pltpu.make_async_copy(src_hbm.at[pl.ds(0, ib)], rows_v, sem_g).wait()

    def drain_out(acc_v):
      pltpu.make_async_copy(acc_v, out_hbm.at[pl.ds(out0, block)],
                            sem_o).wait()

    def reduce_block(rows_v, acc_v):
      @plsc.parallel_loop(0, block, unroll=4)
      def sum_row(i):
        for c in range(d // _LANES):
          sl = pl.ds(c * _LANES, _LANES)
          acc = jnp.maximum(rows_v[i, sl], 0.0)
          for j in range(1, k):
            acc = acc + jnp.maximum(rows_v[j * block + i, sl], 0.0)
          acc_v[i, sl] = acc

    fetch_idx(idx0, 0)
    drain_idx(idx0)
    fetch_gather(idx0, rows0)
    fetch_idx(idx1, 1)

    def pair_body(p, _):
      b = 2 * p
      drain_gather(rows0)
      drain_idx(idx1)
      fetch_gather(idx1, rows1)
      @pl.when(p + 1 < npair)
      def _():
        fetch_idx(idx0, b + 2)
      @pl.when(p >= 1)
      def _():
        drain_out(acc0)
      reduce_block(rows0, acc0)
      pltpu.async_copy(acc0, out_hbm.at[pl.ds(out0 + b * block, block)],
                       sem_o)
      drain_gather(rows1)
      @pl.when(p + 1 < npair)
      def _():
        drain_idx(idx0)
        fetch_gather(idx0, rows0)
        fetch_idx(idx1, b + 3)
      @pl.when(p >= 1)
      def _():
        drain_out(acc1)
      reduce_block(rows1, acc1)
      pltpu.async_copy(acc1, out_hbm.at[pl.ds(out0 + (b + 1) * block, block)],
                       sem_o)
      return 0

    lax.fori_loop(0, npair, pair_body, 0, unroll=False)
    drain_out(acc0)
    drain_out(acc1)

  return gsum


def _mm_t(xt, w, bm):
  kdim, m = xt.shape
  n = w.shape[1]

  def body(x_ref, w_ref, lin_ref):
    lin_ref[...] = jax.lax.dot_general(x_ref[...], w_ref[...],
                                       (((0,), (0,)), ((), ())),
                                       preferred_element_type=jnp.float32)

  return pl.pallas_call(
      body,
      grid=(m // bm,),
      in_specs=[
          pl.BlockSpec((kdim, bm), lambda i: (0, i)),
          pl.BlockSpec((kdim, n), lambda i: (0, 0)),
      ],
      out_specs=pl.BlockSpec((bm, n), lambda i: (i, 0)),
      out_shape=jax.ShapeDtypeStruct((m, n), jnp.float32),
  )(xt, w)


def _update(nei, binput, w, bm):
  m, n = nei.shape

  def body(nei_ref, bin_ref, w_ref, out_ref):
    h = jnp.dot(nei_ref[...], w_ref[...], preferred_element_type=jnp.float32)
    out_ref[...] = bin_ref[...] + h

  return pl.pallas_call(
      body,
      grid=(m // bm,),
      in_specs=[
          pl.BlockSpec((bm, n), lambda i: (i, 0)),
          pl.BlockSpec((bm, n), lambda i: (i, 0)),
          pl.BlockSpec((n, n), lambda i: (0, 0)),
      ],
      out_specs=pl.BlockSpec((bm, n), lambda i: (i, 0)),
      out_shape=jax.ShapeDtypeStruct((m, n), jnp.float32),
  )(nei, binput, w)


def _atom_layer(fatoms, nei, w_o, b_o):
  m, da = fatoms.shape
  n = w_o.shape[1]

  def body(fa_ref, nei_ref, wo_ref, bo_ref, out_ref):
    h = jnp.dot(fa_ref[...], wo_ref[0:da, :], preferred_element_type=jnp.float32)
    h = h + jnp.dot(nei_ref[...], wo_ref[da:, :],
                    preferred_element_type=jnp.float32)
    out_ref[...] = jnp.maximum(h + bo_ref[...], 0.0)

  return pl.pallas_call(
      body,
      out_shape=jax.ShapeDtypeStruct((m, n), jnp.float32),
  )(fatoms, nei, w_o, b_o)


def kernel(fatoms, fbonds, agraph, bgraph, W_i, W_h, W_o, b_o):
  n_atoms, _ = fatoms.shape
  n_bonds, _ = fbonds.shape
  depth = 3

  bond_blk = 40
  atom_blk = 32
  a_pad = -(-n_atoms // (_NW * atom_blk)) * (_NW * atom_blk)

  fatoms_p = jnp.pad(fatoms, ((0, a_pad - n_atoms), (0, 0)))
  bidx = bgraph.T.reshape(-1)
  aidx = jnp.pad(agraph, ((0, a_pad - n_atoms), (0, 0))).T.reshape(-1)

  gsum_bond = _make_gather_sum(n_bonds, bond_blk, _D, _K, chunk=80)
  gsum_atom = _make_gather_sum(a_pad, atom_blk, _D, _K, chunk=96)

  h = _mm_t(fbonds.T, W_i, bm=1280)
  binput = h
  for _ in range(depth - 1):
    nei = gsum_bond(h, bidx)
    h = _update(nei, binput, W_h, bm=2000)
  nei_atoms = gsum_atom(h, aidx)
  atom_h = _atom_layer(fatoms_p, nei_atoms, W_o, b_o.reshape(1, -1))
  return atom_h[:n_atoms]
